# baseline (speedup 1.0000x reference)
import functools

import jax
import jax.numpy as jnp
from jax import lax
from jax.experimental import pallas as pl
from jax.experimental.pallas import tpu as pltpu
from jax.experimental.pallas import tpu_sc as plsc

DIM = 64
B = 4096 * 50
NW = 32
B_PER_W = B // NW
CHUNK = 128
N_CHUNKS = B_PER_W // CHUNK

_mesh = plsc.VectorSubcoreMesh(core_axis_name="c", subcore_axis_name="s")


@functools.partial(
    pl.kernel,
    mesh=_mesh,
    out_type=jax.ShapeDtypeStruct((B, DIM), jnp.float32),
    scratch_types=[
        pltpu.VMEM((N_CHUNKS, CHUNK), jnp.int32),
        pltpu.VMEM((2, CHUNK, DIM), jnp.float32),
        pltpu.SemaphoreType.DMA,
        pltpu.SemaphoreType.DMA,
    ],
    compiler_params=pltpu.CompilerParams(use_tc_tiling_on_sc=False),
)
def _sc_gather(idx_hbm, table_hbm, out_hbm, idx_v, rows_v, gsem, ssem):
    wid = lax.axis_index("s") * 2 + lax.axis_index("c")
    row0 = wid * N_CHUNKS

    pltpu.sync_copy(idx_hbm.at[wid], idx_v)

    pltpu.async_copy(table_hbm.at[idx_v.at[0]], rows_v.at[0], gsem)

    def step(j, _):
        buf = lax.rem(j, 2)
        nbuf = 1 - buf
        @pl.when(j + 1 < N_CHUNKS)
        def _():
            pltpu.async_copy(
                table_hbm.at[idx_v.at[j + 1]], rows_v.at[nbuf], gsem
            )

        pltpu.make_async_copy(
            table_hbm.at[idx_v.at[j]], rows_v.at[buf], gsem
        ).wait()
        out_slice = out_hbm.at[pl.ds((row0 + j) * CHUNK, CHUNK), :]
        pltpu.async_copy(rows_v.at[buf], out_slice, ssem)
        @pl.when(j >= 1)
        def _():
            prev = lax.rem(j + 1, 2)
            prev_slice = out_hbm.at[pl.ds((row0 + j - 1) * CHUNK, CHUNK), :]
            pltpu.make_async_copy(rows_v.at[prev], prev_slice, ssem).wait()
        return _

    lax.fori_loop(0, N_CHUNKS, step, None)
    last = N_CHUNKS - 1
    pltpu.make_async_copy(
        rows_v.at[lax.rem(last, 2)],
        out_hbm.at[pl.ds((row0 + last) * CHUNK, CHUNK), :],
        ssem,
    ).wait()


def kernel(token_ids, weight):
    idx = token_ids.reshape(NW, N_CHUNKS, CHUNK).astype(jnp.int32)
    out = _sc_gather(idx, weight)
    return out.reshape(token_ids.shape + (DIM,))
